# Initial kernel scaffold; baseline (speedup 1.0000x reference)
import jax
import jax.numpy as jnp
from jax.experimental import pallas as pl


def kernel(f, t):
    raise NotImplementedError("write your pallas kernel here")


# speedup vs baseline: 5.3907x; 5.3907x over previous
import jax
import jax.numpy as jnp
from jax.experimental import pallas as pl
from jax.experimental.pallas import tpu as pltpu

_CB = 8


def _pool_body(fe_ref, fo_ref, t_ref, o_ref):
    fe = fe_ref[0]
    fo = fo_ref[0]
    cb, m = fe.shape
    inv = 1.0 / (4.0 * t_ref[:, 0:1])

    def shifted(x, e):
        if e == 0:
            return x
        pad = jnp.full((cb, abs(e)), -jnp.inf, jnp.float32)
        if e > 0:
            return jnp.concatenate([x[:, e:], pad], axis=1)
        return jnp.concatenate([pad, x[:, :e]], axis=1)

    acc = fe + (-0.0) * inv
    for e in range(-6, 7):
        if e != 0:
            acc = jnp.maximum(acc, shifted(fe, e) + (-4.0 * e * e) * inv)
    for e in range(-6, 6):
        w = -float((2 * e + 1) ** 2)
        acc = jnp.maximum(acc, shifted(fo, e) + w * inv)
    o_ref[0] = acc


@jax.jit
def kernel(f, t):
    B, C, N = f.shape
    M = N // 2
    fr = f.reshape(B, C, M, 2)
    fe = fr[..., 0]
    fo = fr[..., 1]
    tb = jnp.broadcast_to(t[:, None], (C, 128))
    grid = (B, C // _CB)
    return pl.pallas_call(
        _pool_body,
        grid=grid,
        in_specs=[
            pl.BlockSpec((1, _CB, M), lambda b, c: (b, c, 0)),
            pl.BlockSpec((1, _CB, M), lambda b, c: (b, c, 0)),
            pl.BlockSpec((_CB, 128), lambda b, c: (c, 0)),
        ],
        out_specs=pl.BlockSpec((1, _CB, M), lambda b, c: (b, c, 0)),
        out_shape=jax.ShapeDtypeStruct((B, C, M), jnp.float32),
        compiler_params=pltpu.CompilerParams(
            dimension_semantics=("parallel", "parallel"),
        ),
    )(fe, fo, tb)


# speedup vs baseline: 5.4319x; 1.0076x over previous
import jax
import jax.numpy as jnp
from jax.experimental import pallas as pl
from jax.experimental.pallas import tpu as pltpu

_CB = 8
_CH = 2048
_M = 8192


def _pool_body(fe_ref, fo_ref, t_ref, o_ref):
    inv = 1.0 / (4.0 * t_ref[:, 0:1])

    def tap(ref, k, e):
        lo, hi = k + e, k + e + _CH
        if 0 <= lo and hi <= _M:
            return ref[0, :, lo:hi]
        clo, chi = max(lo, 0), min(hi, _M)
        x = ref[0, :, clo:chi]
        parts = []
        if clo > lo:
            parts.append(jnp.full((_CB, clo - lo), -jnp.inf, jnp.float32))
        parts.append(x)
        if hi > chi:
            parts.append(jnp.full((_CB, hi - chi), -jnp.inf, jnp.float32))
        return jnp.concatenate(parts, axis=1)

    for k in range(0, _M, _CH):
        acc = tap(fe_ref, k, 0) + (-0.0) * inv
        for e in range(-6, 7):
            if e != 0:
                acc = jnp.maximum(acc, tap(fe_ref, k, e) + (-4.0 * e * e) * inv)
        for e in range(-6, 6):
            w = -float((2 * e + 1) ** 2)
            acc = jnp.maximum(acc, tap(fo_ref, k, e) + w * inv)
        o_ref[0, :, k:k + _CH] = acc


@jax.jit
def kernel(f, t):
    B, C, N = f.shape
    M = N // 2
    fr = f.reshape(B, C, M, 2)
    fe = fr[..., 0]
    fo = fr[..., 1]
    tb = jnp.broadcast_to(t[:, None], (C, 128))
    grid = (B, C // _CB)
    return pl.pallas_call(
        _pool_body,
        grid=grid,
        in_specs=[
            pl.BlockSpec((1, _CB, M), lambda b, c: (b, c, 0)),
            pl.BlockSpec((1, _CB, M), lambda b, c: (b, c, 0)),
            pl.BlockSpec((_CB, 128), lambda b, c: (c, 0)),
        ],
        out_specs=pl.BlockSpec((1, _CB, M), lambda b, c: (b, c, 0)),
        out_shape=jax.ShapeDtypeStruct((B, C, M), jnp.float32),
        compiler_params=pltpu.CompilerParams(
            dimension_semantics=("parallel", "parallel"),
        ),
    )(fe, fo, tb)
